# baseline (speedup 1.0000x reference)
import functools

import numpy as np

import jax
import jax.numpy as jnp
from jax import lax
from jax.experimental import pallas as pl
from jax.experimental.pallas import tpu as pltpu
from jax.experimental.pallas import tpu_sc as plsc

_D = 8192
_CB = 512
_NBLK = _D // _CB
_L = 16

_LANE_NP = np.arange(_L, dtype=np.int32)
_PERM_NP = np.concatenate([_LANE_NP ^ 8, _LANE_NP ^ 4, _LANE_NP ^ 2,
                           _LANE_NP ^ 1])


_sc_mesh = plsc.VectorSubcoreMesh(core_axis_name="c", subcore_axis_name="s")


@functools.partial(
    pl.kernel,
    out_type=jax.ShapeDtypeStruct((_L,), jnp.int32),
    mesh=_sc_mesh,
    scratch_types=[
        pltpu.VMEM((_D,), jnp.float32),
        pltpu.VMEM((_L,), jnp.int32),
        pltpu.VMEM((4 * _L,), jnp.int32),
        pltpu.VMEM((_L,), jnp.int32),
    ],
)
def _sc_argmin(wu_hbm, lane_hbm, perm_hbm, out_hbm, wu_v, lane_v, perm_v,
               out_v):
    wid = lax.axis_index("s") * 2 + lax.axis_index("c")

    @pl.when(wid == 0)
    def _():
        pltpu.sync_copy(wu_hbm, wu_v)
        pltpu.sync_copy(lane_hbm, lane_v)
        pltpu.sync_copy(perm_hbm, perm_v)
        lane = lane_v[...]

        def body(i, carry):
            mv, mi = carry
            v = wu_v[pl.ds(i * _L, _L)]
            idx = i * _L + lane
            pred = v < mv
            return jnp.where(pred, v, mv), jnp.where(pred, idx, mi)

        mv, mi = lax.fori_loop(
            0, _D // _L, body,
            (jnp.broadcast_to(jnp.float32(jnp.inf), (_L,)),
             jnp.broadcast_to(jnp.int32(0), (_L,))))
        for s in range(4):
            pidx = perm_v[pl.ds(s * _L, _L)]
            gv = mv.at[pidx].get(mode="promise_in_bounds",
                                 unique_indices=True)
            gi = mi.at[pidx].get(mode="promise_in_bounds",
                                 unique_indices=True)
            take = (gv < mv) | ((gv == mv) & (gi < mi))
            mv = jnp.where(take, gv, mv)
            mi = jnp.where(take, gi, mi)
        out_v[...] = mi
        pltpu.sync_copy(out_v, out_hbm)


def _flash_body(mi_ref, k_ref, mk_ref, out_ref, acc_ref, m_ref, l_ref,
                nk_ref):
    j = pl.program_id(0)

    @pl.when(j == 0)
    def _init():
        kv = k_ref[...]
        nk_ref[0, 0] = jnp.sqrt(jnp.sum(kv * kv))
        m_ref[0, 0] = -jnp.inf
        l_ref[0, 0] = 0.0

    blk = mk_ref[...]
    kv = k_ref[...]
    cs = jnp.sum(blk * blk, axis=0, keepdims=True)
    dt = lax.dot_general(kv, blk, (((1,), (0,)), ((), ())),
                         preferred_element_type=jnp.float32)
    sim = dt / (nk_ref[0, 0] * jnp.sqrt(cs))
    m_old = m_ref[0, 0]
    m_new = jnp.maximum(m_old, jnp.max(sim))
    p = jnp.exp(sim - m_new)
    scale = jnp.exp(m_old - m_new)
    l_ref[0, 0] = l_ref[0, 0] * scale + jnp.sum(p)
    m_ref[0, 0] = m_new
    col = j * _CB + lax.broadcasted_iota(jnp.int32, (1, _CB), 1)
    pz = jnp.where(col == mi_ref[0, 0], 0.0, p)
    contrib = lax.dot_general(blk, pz, (((1,), (1,)), ((), ())),
                              preferred_element_type=jnp.float32)

    @pl.when(j == 0)
    def _first():
        acc_ref[...] = contrib

    @pl.when(j > 0)
    def _rest():
        acc_ref[...] = acc_ref[...] * scale + contrib

    @pl.when(j == _NBLK - 1)
    def _fin():
        out_ref[...] = acc_ref[...] / l_ref[0, 0]


def kernel(k, u, memory_knowledge, memory_understanding, w_w, w_u, w_lu,
           beta_param):
    mi_vec = _sc_argmin(w_u, jnp.asarray(_LANE_NP), jnp.asarray(_PERM_NP))
    min_idx = mi_vec[0:1].reshape(1, 1)
    k2 = k.reshape(1, _D)
    out = pl.pallas_call(
        _flash_body,
        grid=(_NBLK,),
        in_specs=[
            pl.BlockSpec(memory_space=pltpu.SMEM),
            pl.BlockSpec((1, _D), lambda j: (0, 0)),
            pl.BlockSpec((_D, _CB), lambda j: (0, j)),
        ],
        out_specs=pl.BlockSpec((_D, 1), lambda j: (0, 0)),
        out_shape=jax.ShapeDtypeStruct((_D, 1), jnp.float32),
        scratch_shapes=[
            pltpu.VMEM((_D, 1), jnp.float32),
            pltpu.SMEM((1, 1), jnp.float32),
            pltpu.SMEM((1, 1), jnp.float32),
            pltpu.SMEM((1, 1), jnp.float32),
        ],
        compiler_params=pltpu.CompilerParams(
            dimension_semantics=("arbitrary",),
        ),
    )(min_idx, k2, memory_knowledge)
    return out.reshape(1, _D)


# speedup vs baseline: 1.0174x; 1.0174x over previous
import functools

import numpy as np

import jax
import jax.numpy as jnp
from jax import lax
from jax.experimental import pallas as pl
from jax.experimental.pallas import tpu as pltpu
from jax.experimental.pallas import tpu_sc as plsc

_D = 8192
_CB = 512
_NBLK = _D // _CB
_L = 16

_LANE_NP = np.arange(_L, dtype=np.int32)
_PERM_NP = np.concatenate([_LANE_NP ^ 8, _LANE_NP ^ 4, _LANE_NP ^ 2,
                           _LANE_NP ^ 1])


_sc_mesh = plsc.VectorSubcoreMesh(core_axis_name="c", subcore_axis_name="s")


@functools.partial(
    pl.kernel,
    out_type=jax.ShapeDtypeStruct((_L,), jnp.int32),
    mesh=_sc_mesh,
    scratch_types=[
        pltpu.VMEM((_D,), jnp.float32),
        pltpu.VMEM((_L,), jnp.int32),
        pltpu.VMEM((4 * _L,), jnp.int32),
        pltpu.VMEM((_L,), jnp.int32),
    ],
)
def _sc_argmin(wu_hbm, lane_hbm, perm_hbm, out_hbm, wu_v, lane_v, perm_v,
               out_v):
    wid = lax.axis_index("s") * 2 + lax.axis_index("c")

    @pl.when(wid == 0)
    def _():
        pltpu.sync_copy(wu_hbm, wu_v)
        pltpu.sync_copy(lane_hbm, lane_v)
        pltpu.sync_copy(perm_hbm, perm_v)
        lane = lane_v[...]

        _UNROLL = 8

        def body(i, carry):
            mv, mi = carry
            base = i * (_UNROLL * _L)
            for uu in range(_UNROLL):
                v = wu_v[pl.ds(base + uu * _L, _L)]
                idx = base + uu * _L + lane
                pred = v < mv
                mv = jnp.where(pred, v, mv)
                mi = jnp.where(pred, idx, mi)
            return mv, mi

        mv, mi = lax.fori_loop(
            0, _D // (_UNROLL * _L), body,
            (jnp.broadcast_to(jnp.float32(jnp.inf), (_L,)),
             jnp.broadcast_to(jnp.int32(0), (_L,))))
        for s in range(4):
            pidx = perm_v[pl.ds(s * _L, _L)]
            gv = mv.at[pidx].get(mode="promise_in_bounds",
                                 unique_indices=True)
            gi = mi.at[pidx].get(mode="promise_in_bounds",
                                 unique_indices=True)
            take = (gv < mv) | ((gv == mv) & (gi < mi))
            mv = jnp.where(take, gv, mv)
            mi = jnp.where(take, gi, mi)
        out_v[...] = mi
        pltpu.sync_copy(out_v, out_hbm)


def _flash_body(mi_ref, k_ref, mk_ref, out_ref, acc_ref, m_ref, l_ref,
                nk_ref):
    j = pl.program_id(0)

    @pl.when(j == 0)
    def _init():
        kv = k_ref[...]
        nk_ref[0, 0] = jnp.sqrt(jnp.sum(kv * kv))
        m_ref[0, 0] = -jnp.inf
        l_ref[0, 0] = 0.0

    blk = mk_ref[...]
    kv = k_ref[...]
    cs = jnp.sum(blk * blk, axis=0, keepdims=True)
    dt = lax.dot_general(kv, blk, (((1,), (0,)), ((), ())),
                         preferred_element_type=jnp.float32)
    sim = dt / (nk_ref[0, 0] * jnp.sqrt(cs))
    m_old = m_ref[0, 0]
    m_new = jnp.maximum(m_old, jnp.max(sim))
    p = jnp.exp(sim - m_new)
    scale = jnp.exp(m_old - m_new)
    l_ref[0, 0] = l_ref[0, 0] * scale + jnp.sum(p)
    m_ref[0, 0] = m_new
    col = j * _CB + lax.broadcasted_iota(jnp.int32, (1, _CB), 1)
    pz = jnp.where(col == mi_ref[0], 0.0, p)
    contrib = lax.dot_general(blk, pz, (((1,), (1,)), ((), ())),
                              preferred_element_type=jnp.float32)

    @pl.when(j == 0)
    def _first():
        acc_ref[...] = contrib

    @pl.when(j > 0)
    def _rest():
        acc_ref[...] = acc_ref[...] * scale + contrib

    @pl.when(j == _NBLK - 1)
    def _fin():
        out_ref[...] = acc_ref[...] / l_ref[0, 0]


def kernel(k, u, memory_knowledge, memory_understanding, w_w, w_u, w_lu,
           beta_param):
    mi_vec = _sc_argmin(w_u, jnp.asarray(_LANE_NP), jnp.asarray(_PERM_NP))
    k2 = k.reshape(1, _D)
    out = pl.pallas_call(
        _flash_body,
        grid=(_NBLK,),
        in_specs=[
            pl.BlockSpec(memory_space=pltpu.SMEM),
            pl.BlockSpec((1, _D), lambda j: (0, 0)),
            pl.BlockSpec((_D, _CB), lambda j: (0, j)),
        ],
        out_specs=pl.BlockSpec((_D, 1), lambda j: (0, 0)),
        out_shape=jax.ShapeDtypeStruct((_D, 1), jnp.float32),
        scratch_shapes=[
            pltpu.VMEM((_D, 1), jnp.float32),
            pltpu.SMEM((1, 1), jnp.float32),
            pltpu.SMEM((1, 1), jnp.float32),
            pltpu.SMEM((1, 1), jnp.float32),
        ],
        compiler_params=pltpu.CompilerParams(
            dimension_semantics=("arbitrary",),
        ),
    )(mi_vec, k2, memory_knowledge)
    return out.reshape(1, _D)
